# baseline (speedup 1.0000x reference)
import functools

import jax
import jax.numpy as jnp
from jax.experimental import pallas as pl
from jax.experimental.pallas import tpu as pltpu
from jax.experimental.pallas import tpu_sc as plsc

DIM = 1024
LATENT = 512
NE = 8
TOKENS = 2048
TB = 512
BLK = 128
NBLK = 40
CAP = NBLK * BLK
HMAX = 2048
NW = 32
TPT = TOKENS // NW
COST_LAMBDA = 5e-07
_SQRT_HALF = 0.7071067811865476


def _gelu(x):
    x = x.astype(jnp.float32)
    return x * 0.5 * (1.0 + jax.lax.erf(x * _SQRT_HALF))


def _bdot(a16, b16):
    return jax.lax.dot_general(
        a16, b16, (((1,), (1,)), ((), ())), preferred_element_type=jnp.float32)


def _stage1_body(x_ref, wdown_ref, wrhi_ref, wrlo_ref, brow_ref, wcore_ref,
                 ws1_ref, ws2_ref, bdown_ref, bcore_ref, bs1_ref, bs2_ref,
                 g_ref, wg_ref, counts_ref, sh_ref, core_ref):
    xb = x_ref[...]
    xb16 = xb.astype(jnp.bfloat16)
    xlo16 = (xb - xb16.astype(jnp.float32)).astype(jnp.bfloat16)

    h = _bdot(xb16, wdown_ref[...]) + bdown_ref[...]
    g = _gelu(h)
    g16 = g.astype(jnp.bfloat16)
    g_ref[...] = g16

    logits = (_bdot(xb16, wrhi_ref[...])
              + (_bdot(xb16, wrlo_ref[...]) + _bdot(xlo16, wrhi_ref[...]))
              + brow_ref[...])
    mx = jnp.max(logits, axis=-1, keepdims=True)
    ex = jnp.exp(logits - mx)
    probs = ex / jnp.sum(ex, axis=-1, keepdims=True)

    iota = jax.lax.broadcasted_iota(jnp.int32, probs.shape, 1)
    m0 = jnp.max(probs, axis=-1, keepdims=True)
    e0 = jnp.min(jnp.where(probs >= m0, iota, NE), axis=-1, keepdims=True)
    oh0 = (iota == e0)
    probs1 = jnp.where(oh0, -1.0, probs)
    m1 = jnp.max(probs1, axis=-1, keepdims=True)
    e1 = jnp.min(jnp.where(probs1 >= m1, iota, NE), axis=-1, keepdims=True)
    oh1 = (iota == e1)

    ed = jnp.exp(m1 - m0)
    gate0 = 1.0 / (1.0 + ed)
    gate1 = ed * gate0

    w = gate0 * oh0.astype(jnp.float32) + gate1 * oh1.astype(jnp.float32)
    wg_ref[...] = jnp.concatenate(
        [w, jnp.broadcast_to(gate0, (TB, 4)), jnp.broadcast_to(gate1, (TB, 4))],
        axis=-1)

    a0 = jnp.max(oh0.astype(jnp.float32), axis=0, keepdims=True)
    a1 = jnp.max(oh1.astype(jnp.float32), axis=0, keepdims=True)
    cblk = jnp.broadcast_to(jnp.concatenate([a0, a1], axis=-1), (8, 2 * NE))

    @pl.when(pl.program_id(0) == 0)
    def _init():
        counts_ref[...] = cblk

    @pl.when(pl.program_id(0) > 0)
    def _acc():
        counts_ref[...] = jnp.maximum(counts_ref[...], cblk)

    core_ref[...] = _bdot(_gelu(xb).astype(jnp.bfloat16), wcore_ref[...]) \
        + bcore_ref[...]

    s1 = _gelu(_bdot(g16, ws1_ref[...]) + bs1_ref[...])
    sh_ref[...] = (_bdot(s1.astype(jnp.bfloat16), ws2_ref[...])
                   + bs2_ref[...]).astype(jnp.bfloat16)


def _routing_body(wg_ref, dg_ref, bmap_ref):
    w = wg_ref[:, 0:NE]
    sel = (w > 0.0).astype(jnp.float32)
    iota = jax.lax.broadcasted_iota(jnp.int32, w.shape, 1)
    ia = jnp.min(jnp.where(sel > 0.5, iota, NE), axis=-1, keepdims=True)
    oha = (iota == ia)
    selb = jnp.where(oha, 0.0, sel)
    ib = jnp.min(jnp.where(selb > 0.5, iota, NE), axis=-1, keepdims=True)
    ohb = (iota == ib)
    ohaf = oha.astype(jnp.float32)
    ohbf = ohb.astype(jnp.float32)

    ga = jnp.sum(w * ohaf, axis=-1, keepdims=True)
    gb = jnp.sum(w * ohbf, axis=-1, keepdims=True)

    cum = sel
    shift = 1
    while shift < TOKENS:
        cum = cum + jnp.concatenate(
            [jnp.zeros((shift, NE), jnp.float32), cum[:TOKENS - shift, :]],
            axis=0)
        shift *= 2
    excl = cum - sel
    cnt = cum[TOKENS - 1:TOKENS, :]
    nb = jnp.floor((cnt + (BLK - 1)) * (1.0 / BLK))
    tri = (jax.lax.broadcasted_iota(jnp.int32, (NE, NE), 0)
           <= jax.lax.broadcasted_iota(jnp.int32, (NE, NE), 1))
    cumblk = jax.lax.dot_general(
        nb, tri.astype(jnp.float32), (((1,), (0,)), ((), ())),
        preferred_element_type=jnp.float32)
    off = (cumblk - nb) * float(BLK)

    da = (jnp.sum(excl * ohaf, axis=-1, keepdims=True)
          + jnp.sum(off * ohaf, axis=-1, keepdims=True))
    db = (jnp.sum(excl * ohbf, axis=-1, keepdims=True)
          + jnp.sum(off * ohbf, axis=-1, keepdims=True))
    dg_ref[...] = jnp.concatenate(
        [da, db, ga, gb, jnp.zeros((TOKENS, 4), jnp.float32)], axis=-1)

    jblk = jax.lax.broadcasted_iota(
        jnp.int32, (1, 64), 1).astype(jnp.float32)
    bacc = jnp.zeros((1, 64), jnp.float32)
    for e in range(NE):
        bacc = bacc + (jblk >= cumblk[0:1, e:e + 1]).astype(jnp.float32)
    bmap_ref[...] = jnp.broadcast_to(jnp.minimum(bacc, float(NE - 1)), (8, 64))


def _sc_dispatch(g_i32, da, db):
    mesh = plsc.VectorSubcoreMesh(core_axis_name="c", subcore_axis_name="s")

    @functools.partial(
        pl.kernel, mesh=mesh,
        out_type=jax.ShapeDtypeStruct((CAP, LATENT // 2), jnp.int32),
        scratch_types=[
            pltpu.VMEM((TPT, LATENT // 2), jnp.int32),
            pltpu.VMEM((TPT,), jnp.int32),
            pltpu.VMEM((TPT,), jnp.int32),
            pltpu.SemaphoreType.DMA,
            pltpu.SemaphoreType.DMA,
        ],
    )
    def disp(g_hbm, da_hbm, db_hbm, buf_hbm, rows_v, da_v, db_v, sem0, sem1):
        wid = jax.lax.axis_index("s") * 2 + jax.lax.axis_index("c")
        base = wid * TPT
        pltpu.sync_copy(da_hbm.at[pl.ds(base, TPT)], da_v)
        pltpu.sync_copy(db_hbm.at[pl.ds(base, TPT)], db_v)
        pltpu.sync_copy(g_hbm.at[pl.ds(base, TPT)], rows_v)
        cp0 = pltpu.async_copy(rows_v, buf_hbm.at[da_v], sem0)
        cp1 = pltpu.async_copy(rows_v, buf_hbm.at[db_v], sem1)
        cp0.wait()
        cp1.wait()

    return disp(g_i32, da, db)


def _sc_gather(y_i32, da, db):
    mesh = plsc.VectorSubcoreMesh(core_axis_name="c", subcore_axis_name="s")

    @functools.partial(
        pl.kernel, mesh=mesh,
        out_type=[jax.ShapeDtypeStruct((TOKENS, LATENT // 2), jnp.int32),
                  jax.ShapeDtypeStruct((TOKENS, LATENT // 2), jnp.int32)],
        scratch_types=[
            pltpu.VMEM((TPT, LATENT // 2), jnp.int32),
            pltpu.VMEM((TPT, LATENT // 2), jnp.int32),
            pltpu.VMEM((TPT,), jnp.int32),
            pltpu.VMEM((TPT,), jnp.int32),
            pltpu.SemaphoreType.DMA,
            pltpu.SemaphoreType.DMA,
        ],
    )
    def gath(y_hbm, da_hbm, db_hbm, ya_hbm, yb_hbm,
             rowsa_v, rowsb_v, da_v, db_v, sem0, sem1):
        wid = jax.lax.axis_index("s") * 2 + jax.lax.axis_index("c")
        base = wid * TPT
        pltpu.sync_copy(da_hbm.at[pl.ds(base, TPT)], da_v)
        pltpu.sync_copy(db_hbm.at[pl.ds(base, TPT)], db_v)
        cp0 = pltpu.async_copy(y_hbm.at[da_v], rowsa_v, sem0)
        cp1 = pltpu.async_copy(y_hbm.at[db_v], rowsb_v, sem1)
        cp0.wait()
        cp1.wait()
        pltpu.sync_copy(rowsa_v, ya_hbm.at[pl.ds(base, TPT)])
        pltpu.sync_copy(rowsb_v, yb_hbm.at[pl.ds(base, TPT)])

    return gath(y_i32, da, db)


def _grouped_body(*refs):
    bm_ref = refs[0]
    buf_ref = refs[1]
    w1_refs = refs[2:2 + NE]
    b1_refs = refs[2 + NE:2 + 2 * NE]
    w2_refs = refs[2 + 2 * NE:2 + 3 * NE]
    b2_refs = refs[2 + 3 * NE:2 + 4 * NE]
    y_ref = refs[-1]

    e = bm_ref[pl.program_id(0)]

    def _mk(ei):
        def _branch():
            z1 = _gelu(_bdot(buf_ref[...], w1_refs[ei][...])
                       + b1_refs[ei][...])
            return _bdot(z1.astype(jnp.bfloat16), w2_refs[ei][...]) \
                + b2_refs[ei][...]
        return _branch

    y_ref[...] = jax.lax.switch(
        e, [_mk(ei) for ei in range(NE)]).astype(jnp.bfloat16)


def _combine_body(*refs):
    (ya_ref, yb_ref, wg_ref, dg_ref, sh_ref, core_ref, counts_ref,
     wup_ref, bup_ref) = refs[:9]
    b1_refs = refs[9:9 + NE]
    w2_refs = refs[9 + NE:9 + 2 * NE]
    b2_refs = refs[9 + 2 * NE:9 + 3 * NE]
    out_ref = refs[-2]
    ca_ref = refs[-1]

    @pl.when(pl.program_id(0) == 0)
    def _constants():
        for e in range(NE):
            gb16 = _gelu(b1_refs[e][...]).astype(jnp.bfloat16)
            ca_ref[e:e + 1, :] = _bdot(gb16, w2_refs[e][...]) + b2_refs[e][...]
        c = ca_ref[0:NE, :]
        act0 = counts_ref[0:1, 0:NE] > 0.5
        act1 = counts_ref[0:1, NE:2 * NE] > 0.5
        ca_ref[NE:NE + 1, :] = jnp.sum(jnp.where(act0.T, c, 0.0), axis=0,
                                       keepdims=True)
        ca_ref[NE + 1:NE + 2, :] = jnp.sum(jnp.where(act1.T, c, 0.0), axis=0,
                                           keepdims=True)

    wg = wg_ref[...]
    dg = dg_ref[...]
    ga = dg[:, 2:3]
    gb = dg[:, 3:4]
    moe = (ga * ya_ref[...].astype(jnp.float32)
           + gb * yb_ref[...].astype(jnp.float32))
    for e in range(NE):
        moe = moe - wg[:, e:e + 1] * ca_ref[e:e + 1, :]
    gate0 = wg[:, NE:NE + 1]
    gate1 = wg[:, NE + 4:NE + 5]
    acc = (moe + gate0 * ca_ref[NE:NE + 1, :]
           + gate1 * ca_ref[NE + 1:NE + 2, :])
    t16 = (acc + 0.1 * sh_ref[...].astype(jnp.float32)).astype(jnp.bfloat16)
    out_ref[...] = _bdot(t16, wup_ref[...]) + bup_ref[...] + core_ref[...]


def kernel(x, W_down, b_down, W_up, b_up, expert_W1s, expert_b1s, expert_W2s,
           expert_b2s, W_shared1, b_shared1, W_shared2, b_shared2, W_router,
           b_router, W_core, b_core, expert_cost):
    f32 = jnp.float32
    bf16 = jnp.bfloat16
    X = x.reshape(TOKENS, DIM)
    nblk = TOKENS // TB

    brow = (b_router - COST_LAMBDA * expert_cost).reshape(1, NE)
    wr_hi = W_router.astype(bf16)
    wr_lo = (W_router - wr_hi.astype(f32)).astype(bf16)

    full = lambda shape: pl.BlockSpec(shape, lambda i: (0,) * len(shape))
    tb = lambda n: pl.BlockSpec((TB, n), lambda i: (i, 0))

    g16, wg, counts, sh16, core = pl.pallas_call(
        _stage1_body,
        grid=(nblk,),
        in_specs=[
            tb(DIM),
            full((LATENT, DIM)), full((NE, DIM)), full((NE, DIM)),
            full((1, NE)),
            full((DIM, DIM)), full((LATENT, LATENT)), full((LATENT, LATENT)),
            full((1, LATENT)), full((1, DIM)), full((1, LATENT)),
            full((1, LATENT)),
        ],
        out_specs=[
            tb(LATENT), tb(2 * NE),
            pl.BlockSpec((8, 2 * NE), lambda i: (0, 0)),
            tb(LATENT), tb(DIM),
        ],
        out_shape=[
            jax.ShapeDtypeStruct((TOKENS, LATENT), bf16),
            jax.ShapeDtypeStruct((TOKENS, 2 * NE), f32),
            jax.ShapeDtypeStruct((8, 2 * NE), f32),
            jax.ShapeDtypeStruct((TOKENS, LATENT), bf16),
            jax.ShapeDtypeStruct((TOKENS, DIM), f32),
        ],
    )(X, W_down.astype(bf16), wr_hi, wr_lo, brow, W_core.astype(bf16),
      W_shared1.astype(bf16), W_shared2.astype(bf16),
      b_down.reshape(1, LATENT), b_core.reshape(1, DIM),
      b_shared1.reshape(1, LATENT), b_shared2.reshape(1, LATENT))

    dg, bmapf = pl.pallas_call(
        _routing_body,
        grid=(1,),
        in_specs=[full((TOKENS, 2 * NE))],
        out_specs=[full((TOKENS, NE)), full((8, 64))],
        out_shape=[jax.ShapeDtypeStruct((TOKENS, NE), f32),
                   jax.ShapeDtypeStruct((8, 64), f32)],
    )(wg)

    da = dg[:, 0].astype(jnp.int32)
    db = dg[:, 1].astype(jnp.int32)
    bmap = bmapf[0, 0:NBLK].astype(jnp.int32)

    g_i32 = jax.lax.bitcast_convert_type(
        g16.reshape(TOKENS, LATENT // 2, 2), jnp.int32)
    buf_i32 = jnp.zeros((CAP, LATENT // 2), jnp.int32).at[da].set(
        g_i32).at[db].set(g_i32)
    buf16 = jax.lax.bitcast_convert_type(buf_i32, bf16).reshape(CAP, LATENT)

    gspecs = [pl.BlockSpec((BLK, LATENT), lambda i, bm: (i, 0))]
    gargs = [buf16]
    fullp = lambda shape: pl.BlockSpec(shape, lambda i, bm: (0,) * len(shape))
    for e in range(NE):
        hd = expert_W1s[e].shape[0]
        gspecs.append(fullp((hd, LATENT)))
        gargs.append(expert_W1s[e].astype(bf16))
    for e in range(NE):
        hd = expert_b1s[e].shape[0]
        gspecs.append(fullp((1, hd)))
        gargs.append(expert_b1s[e].reshape(1, hd))
    for e in range(NE):
        hd = expert_W2s[e].shape[1]
        gspecs.append(fullp((LATENT, hd)))
        gargs.append(expert_W2s[e].astype(bf16))
    for e in range(NE):
        gspecs.append(fullp((1, LATENT)))
        gargs.append(expert_b2s[e].reshape(1, LATENT))

    y16 = pl.pallas_call(
        _grouped_body,
        grid_spec=pltpu.PrefetchScalarGridSpec(
            num_scalar_prefetch=1,
            grid=(NBLK,),
            in_specs=gspecs,
            out_specs=pl.BlockSpec((BLK, LATENT), lambda i, bm: (i, 0)),
        ),
        out_shape=jax.ShapeDtypeStruct((CAP, LATENT), bf16),
    )(bmap, *gargs)

    y_i32 = jax.lax.bitcast_convert_type(
        y16.reshape(CAP, LATENT // 2, 2), jnp.int32)
    ya_i32, yb_i32 = y_i32[da], y_i32[db]
    ya16 = jax.lax.bitcast_convert_type(ya_i32, bf16).reshape(TOKENS, LATENT)
    yb16 = jax.lax.bitcast_convert_type(yb_i32, bf16).reshape(TOKENS, LATENT)

    in_specs5 = [tb(LATENT), tb(LATENT), tb(2 * NE), tb(NE), tb(LATENT),
                 tb(DIM), full((8, 2 * NE)), full((DIM, LATENT)),
                 full((1, DIM))]
    args5 = [ya16, yb16, wg, dg, sh16, core, counts, W_up.astype(bf16),
             b_up.reshape(1, DIM)]
    for e in range(NE):
        hd = expert_b1s[e].shape[0]
        in_specs5.append(full((1, hd)))
        args5.append(expert_b1s[e].reshape(1, hd))
    for e in range(NE):
        hd = expert_W2s[e].shape[1]
        in_specs5.append(full((LATENT, hd)))
        args5.append(expert_W2s[e].astype(bf16))
    for e in range(NE):
        in_specs5.append(full((1, LATENT)))
        args5.append(expert_b2s[e].reshape(1, LATENT))

    out = pl.pallas_call(
        _combine_body,
        grid=(nblk,),
        in_specs=in_specs5,
        out_specs=tb(DIM),
        out_shape=jax.ShapeDtypeStruct((TOKENS, DIM), f32),
        scratch_shapes=[pltpu.VMEM((2 * NE, LATENT), f32)],
    )(*args5)

    return out.reshape(x.shape)


# speedup vs baseline: 2.7334x; 2.7334x over previous
import jax
import jax.numpy as jnp
from jax.experimental import pallas as pl
from jax.experimental.pallas import tpu as pltpu

DIM = 1024
LATENT = 512
NE = 8
TOKENS = 2048
TB = 512
TB3 = 256
COST_LAMBDA = 5e-07
_SQRT_HALF = 0.7071067811865476


def _gelu(x):
    x = x.astype(jnp.float32)
    return x * 0.5 * (1.0 + jax.lax.erf(x * _SQRT_HALF))


def _bdot(a16, b16):
    return jax.lax.dot_general(
        a16, b16, (((1,), (1,)), ((), ())), preferred_element_type=jnp.float32)


def _stage1_body(x_ref, wdown_ref, wrhi_ref, wrlo_ref, brow_ref, wcore_ref,
                 ws1_ref, ws2_ref, bdown_ref, bcore_ref, bs1_ref, bs2_ref,
                 g_ref, wg_ref, counts_ref, sh_ref, core_ref):
    xb = x_ref[...]
    xb16 = xb.astype(jnp.bfloat16)
    xlo16 = (xb - xb16.astype(jnp.float32)).astype(jnp.bfloat16)

    h = _bdot(xb16, wdown_ref[...]) + bdown_ref[...]
    g = _gelu(h)
    g16 = g.astype(jnp.bfloat16)
    g_ref[...] = g16

    logits = (_bdot(xb16, wrhi_ref[...])
              + (_bdot(xb16, wrlo_ref[...]) + _bdot(xlo16, wrhi_ref[...]))
              + brow_ref[...])
    mx = jnp.max(logits, axis=-1, keepdims=True)
    ex = jnp.exp(logits - mx)
    probs = ex / jnp.sum(ex, axis=-1, keepdims=True)

    iota = jax.lax.broadcasted_iota(jnp.int32, probs.shape, 1)
    m0 = jnp.max(probs, axis=-1, keepdims=True)
    e0 = jnp.min(jnp.where(probs >= m0, iota, NE), axis=-1, keepdims=True)
    oh0 = (iota == e0)
    probs1 = jnp.where(oh0, -1.0, probs)
    m1 = jnp.max(probs1, axis=-1, keepdims=True)
    e1 = jnp.min(jnp.where(probs1 >= m1, iota, NE), axis=-1, keepdims=True)
    oh1 = (iota == e1)

    ed = jnp.exp(m1 - m0)
    gate0 = 1.0 / (1.0 + ed)
    gate1 = ed * gate0

    w = gate0 * oh0.astype(jnp.float32) + gate1 * oh1.astype(jnp.float32)
    wg_ref[...] = jnp.concatenate(
        [w, jnp.broadcast_to(gate0, (TB, 4)), jnp.broadcast_to(gate1, (TB, 4))],
        axis=-1)

    a0 = jnp.max(oh0.astype(jnp.float32), axis=0, keepdims=True)
    a1 = jnp.max(oh1.astype(jnp.float32), axis=0, keepdims=True)
    cblk = jnp.broadcast_to(jnp.concatenate([a0, a1], axis=-1), (8, 2 * NE))

    @pl.when(pl.program_id(0) == 0)
    def _init():
        counts_ref[...] = cblk

    @pl.when(pl.program_id(0) > 0)
    def _acc():
        counts_ref[...] = jnp.maximum(counts_ref[...], cblk)

    core_ref[...] = _bdot(_gelu(xb).astype(jnp.bfloat16), wcore_ref[...]) \
        + bcore_ref[...]

    s1 = _gelu(_bdot(g16, ws1_ref[...]) + bs1_ref[...])
    sh_ref[...] = (_bdot(s1.astype(jnp.bfloat16), ws2_ref[...])
                   + bs2_ref[...]).astype(jnp.bfloat16)


def _stage3_body(*refs):
    (g_ref, wg_ref, sh_ref, core_ref, counts_ref, wup_ref, bup_ref) = refs[:7]
    w1_refs = refs[7:7 + NE]
    b1_refs = refs[7 + NE:7 + 2 * NE]
    w2_refs = refs[7 + 2 * NE:7 + 3 * NE]
    b2_refs = refs[7 + 3 * NE:7 + 4 * NE]
    out_ref = refs[-2]
    ca_ref = refs[-1]

    @pl.when(pl.program_id(0) == 0)
    def _constants():
        for e in range(NE):
            gb = _gelu(b1_refs[e][...]).astype(jnp.bfloat16)
            ca_ref[e:e + 1, :] = _bdot(gb, w2_refs[e][...]) + b2_refs[e][...]
        c = ca_ref[0:NE, :]
        act0 = counts_ref[0:1, 0:NE] > 0.5
        act1 = counts_ref[0:1, NE:2 * NE] > 0.5
        ca_ref[NE:NE + 1, :] = jnp.sum(jnp.where(act0.T, c, 0.0), axis=0,
                                       keepdims=True)
        ca_ref[NE + 1:NE + 2, :] = jnp.sum(jnp.where(act1.T, c, 0.0), axis=0,
                                           keepdims=True)

    g16 = g_ref[...]
    wg = wg_ref[...]
    moe = jnp.zeros((TB3, LATENT), jnp.float32)
    for e in range(NE):
        z1 = _gelu(_bdot(g16, w1_refs[e][...]) + b1_refs[e][...])
        z2 = _bdot(z1.astype(jnp.bfloat16), w2_refs[e][...]) + b2_refs[e][...]
        moe = moe + wg[:, e:e + 1] * (z2 - ca_ref[e:e + 1, :])
    gate0 = wg[:, NE:NE + 1]
    gate1 = wg[:, NE + 4:NE + 5]
    acc = (moe + gate0 * ca_ref[NE:NE + 1, :]
           + gate1 * ca_ref[NE + 1:NE + 2, :])
    t16 = (acc + 0.1 * sh_ref[...].astype(jnp.float32)).astype(jnp.bfloat16)
    out_ref[...] = _bdot(t16, wup_ref[...]) + bup_ref[...] + core_ref[...]


def kernel(x, W_down, b_down, W_up, b_up, expert_W1s, expert_b1s, expert_W2s,
           expert_b2s, W_shared1, b_shared1, W_shared2, b_shared2, W_router,
           b_router, W_core, b_core, expert_cost):
    f32 = jnp.float32
    bf16 = jnp.bfloat16
    X = x.reshape(TOKENS, DIM)
    nblk = TOKENS // TB

    brow = (b_router - COST_LAMBDA * expert_cost).reshape(1, NE)
    wr_hi = W_router.astype(bf16)
    wr_lo = (W_router - wr_hi.astype(f32)).astype(bf16)

    full = lambda shape: pl.BlockSpec(shape, lambda i: (0,) * len(shape))
    tb = lambda n: pl.BlockSpec((TB, n), lambda i: (i, 0))

    g16, wg, counts, sh16, core = pl.pallas_call(
        _stage1_body,
        grid=(nblk,),
        in_specs=[
            tb(DIM),
            full((LATENT, DIM)), full((NE, DIM)), full((NE, DIM)),
            full((1, NE)),
            full((DIM, DIM)), full((LATENT, LATENT)), full((LATENT, LATENT)),
            full((1, LATENT)), full((1, DIM)), full((1, LATENT)),
            full((1, LATENT)),
        ],
        out_specs=[
            tb(LATENT), tb(2 * NE),
            pl.BlockSpec((8, 2 * NE), lambda i: (0, 0)),
            tb(LATENT), tb(DIM),
        ],
        out_shape=[
            jax.ShapeDtypeStruct((TOKENS, LATENT), bf16),
            jax.ShapeDtypeStruct((TOKENS, 2 * NE), f32),
            jax.ShapeDtypeStruct((8, 2 * NE), f32),
            jax.ShapeDtypeStruct((TOKENS, LATENT), bf16),
            jax.ShapeDtypeStruct((TOKENS, DIM), f32),
        ],
    )(X, W_down.astype(bf16), wr_hi, wr_lo, brow, W_core.astype(bf16),
      W_shared1.astype(bf16), W_shared2.astype(bf16),
      b_down.reshape(1, LATENT), b_core.reshape(1, DIM),
      b_shared1.reshape(1, LATENT), b_shared2.reshape(1, LATENT))

    tb3 = lambda n: pl.BlockSpec((TB3, n), lambda i: (i, 0))
    in_specs3 = [tb3(LATENT), tb3(2 * NE), tb3(LATENT), tb3(DIM),
                 full((8, 2 * NE)), full((DIM, LATENT)), full((1, DIM))]
    args3 = [g16, wg, sh16, core, counts, W_up.astype(bf16),
             b_up.reshape(1, DIM)]
    for e in range(NE):
        hd = expert_W1s[e].shape[0]
        in_specs3.append(full((hd, LATENT)))
        args3.append(expert_W1s[e].astype(bf16))
    for e in range(NE):
        hd = expert_b1s[e].shape[0]
        in_specs3.append(full((1, hd)))
        args3.append(expert_b1s[e].reshape(1, hd))
    for e in range(NE):
        hd = expert_W2s[e].shape[1]
        in_specs3.append(full((LATENT, hd)))
        args3.append(expert_W2s[e].astype(bf16))
    for e in range(NE):
        in_specs3.append(full((1, LATENT)))
        args3.append(expert_b2s[e].reshape(1, LATENT))

    out = pl.pallas_call(
        _stage3_body,
        grid=(TOKENS // TB3,),
        in_specs=in_specs3,
        out_specs=tb3(DIM),
        out_shape=jax.ShapeDtypeStruct((TOKENS, DIM), f32),
        scratch_shapes=[pltpu.VMEM((2 * NE, LATENT), f32)],
    )(*args3)

    return out.reshape(x.shape)


# speedup vs baseline: 2.8850x; 1.0555x over previous
import jax
import jax.numpy as jnp
from jax.experimental import pallas as pl
from jax.experimental.pallas import tpu as pltpu

DIM = 1024
LATENT = 512
NE = 8
TOKENS = 2048
TB = 512
TB3 = 1024
COST_LAMBDA = 5e-07
_SQRT_HALF = 0.7071067811865476


def _gelu(x):
    x = x.astype(jnp.float32)
    return x * 0.5 * (1.0 + jax.lax.erf(x * _SQRT_HALF))


def _bdot(a16, b16):
    return jax.lax.dot_general(
        a16, b16, (((1,), (1,)), ((), ())), preferred_element_type=jnp.float32)


def _stage1_body(x_ref, wdown_ref, wrhi_ref, wrlo_ref, brow_ref, wcore_ref,
                 ws1_ref, ws2_ref, bdown_ref, bcore_ref, bs1_ref, bs2_ref,
                 g_ref, wg_ref, counts_ref, sh_ref, core_ref):
    xb = x_ref[...]
    xb16 = xb.astype(jnp.bfloat16)
    xlo16 = (xb - xb16.astype(jnp.float32)).astype(jnp.bfloat16)

    h = _bdot(xb16, wdown_ref[...]) + bdown_ref[...]
    g = _gelu(h)
    g16 = g.astype(jnp.bfloat16)
    g_ref[...] = g16

    logits = (_bdot(xb16, wrhi_ref[...])
              + (_bdot(xb16, wrlo_ref[...]) + _bdot(xlo16, wrhi_ref[...]))
              + brow_ref[...])
    mx = jnp.max(logits, axis=-1, keepdims=True)
    ex = jnp.exp(logits - mx)
    probs = ex / jnp.sum(ex, axis=-1, keepdims=True)

    iota = jax.lax.broadcasted_iota(jnp.int32, probs.shape, 1)
    m0 = jnp.max(probs, axis=-1, keepdims=True)
    e0 = jnp.min(jnp.where(probs >= m0, iota, NE), axis=-1, keepdims=True)
    oh0 = (iota == e0)
    probs1 = jnp.where(oh0, -1.0, probs)
    m1 = jnp.max(probs1, axis=-1, keepdims=True)
    e1 = jnp.min(jnp.where(probs1 >= m1, iota, NE), axis=-1, keepdims=True)
    oh1 = (iota == e1)

    ed = jnp.exp(m1 - m0)
    gate0 = 1.0 / (1.0 + ed)
    gate1 = ed * gate0

    w = gate0 * oh0.astype(jnp.float32) + gate1 * oh1.astype(jnp.float32)
    wg_ref[...] = jnp.concatenate(
        [w, jnp.broadcast_to(gate0, (TB, 4)), jnp.broadcast_to(gate1, (TB, 4))],
        axis=-1)

    a0 = jnp.max(oh0.astype(jnp.float32), axis=0, keepdims=True)
    a1 = jnp.max(oh1.astype(jnp.float32), axis=0, keepdims=True)
    cblk = jnp.broadcast_to(jnp.concatenate([a0, a1], axis=-1), (8, 2 * NE))

    @pl.when(pl.program_id(0) == 0)
    def _init():
        counts_ref[...] = cblk

    @pl.when(pl.program_id(0) > 0)
    def _acc():
        counts_ref[...] = jnp.maximum(counts_ref[...], cblk)

    core_ref[...] = _bdot(_gelu(xb).astype(jnp.bfloat16), wcore_ref[...]) \
        + bcore_ref[...]

    s1 = _gelu(_bdot(g16, ws1_ref[...]) + bs1_ref[...])
    sh_ref[...] = (_bdot(s1.astype(jnp.bfloat16), ws2_ref[...])
                   + bs2_ref[...]).astype(jnp.bfloat16)


def _stage3_body(*refs):
    (g_ref, wg_ref, sh_ref, core_ref, counts_ref, wup_ref, bup_ref) = refs[:7]
    w1_refs = refs[7:7 + NE]
    b1_refs = refs[7 + NE:7 + 2 * NE]
    w2_refs = refs[7 + 2 * NE:7 + 3 * NE]
    b2_refs = refs[7 + 3 * NE:7 + 4 * NE]
    out_ref = refs[-2]
    ca_ref = refs[-1]

    @pl.when(pl.program_id(0) == 0)
    def _constants():
        for e in range(NE):
            gb = _gelu(b1_refs[e][...]).astype(jnp.bfloat16)
            ca_ref[e:e + 1, :] = _bdot(gb, w2_refs[e][...]) + b2_refs[e][...]
        c = ca_ref[0:NE, :]
        act0 = counts_ref[0:1, 0:NE] > 0.5
        act1 = counts_ref[0:1, NE:2 * NE] > 0.5
        ca_ref[NE:NE + 1, :] = jnp.sum(jnp.where(act0.T, c, 0.0), axis=0,
                                       keepdims=True)
        ca_ref[NE + 1:NE + 2, :] = jnp.sum(jnp.where(act1.T, c, 0.0), axis=0,
                                           keepdims=True)

    g16 = g_ref[...]
    wg = wg_ref[...]
    moe = jnp.zeros((TB3, LATENT), jnp.float32)
    for e in range(NE):
        z1 = _gelu(_bdot(g16, w1_refs[e][...]) + b1_refs[e][...])
        z2 = _bdot(z1.astype(jnp.bfloat16), w2_refs[e][...]) + b2_refs[e][...]
        moe = moe + wg[:, e:e + 1] * (z2 - ca_ref[e:e + 1, :])
    gate0 = wg[:, NE:NE + 1]
    gate1 = wg[:, NE + 4:NE + 5]
    acc = (moe + gate0 * ca_ref[NE:NE + 1, :]
           + gate1 * ca_ref[NE + 1:NE + 2, :])
    t16 = (acc + 0.1 * sh_ref[...].astype(jnp.float32)).astype(jnp.bfloat16)
    out_ref[...] = _bdot(t16, wup_ref[...]) + bup_ref[...] + core_ref[...]


def kernel(x, W_down, b_down, W_up, b_up, expert_W1s, expert_b1s, expert_W2s,
           expert_b2s, W_shared1, b_shared1, W_shared2, b_shared2, W_router,
           b_router, W_core, b_core, expert_cost):
    f32 = jnp.float32
    bf16 = jnp.bfloat16
    X = x.reshape(TOKENS, DIM)
    nblk = TOKENS // TB

    brow = (b_router - COST_LAMBDA * expert_cost).reshape(1, NE)
    wr_hi = W_router.astype(bf16)
    wr_lo = (W_router - wr_hi.astype(f32)).astype(bf16)

    full = lambda shape: pl.BlockSpec(shape, lambda i: (0,) * len(shape))
    tb = lambda n: pl.BlockSpec((TB, n), lambda i: (i, 0))

    g16, wg, counts, sh16, core = pl.pallas_call(
        _stage1_body,
        grid=(nblk,),
        in_specs=[
            tb(DIM),
            full((LATENT, DIM)), full((NE, DIM)), full((NE, DIM)),
            full((1, NE)),
            full((DIM, DIM)), full((LATENT, LATENT)), full((LATENT, LATENT)),
            full((1, LATENT)), full((1, DIM)), full((1, LATENT)),
            full((1, LATENT)),
        ],
        out_specs=[
            tb(LATENT), tb(2 * NE),
            pl.BlockSpec((8, 2 * NE), lambda i: (0, 0)),
            tb(LATENT), tb(DIM),
        ],
        out_shape=[
            jax.ShapeDtypeStruct((TOKENS, LATENT), bf16),
            jax.ShapeDtypeStruct((TOKENS, 2 * NE), f32),
            jax.ShapeDtypeStruct((8, 2 * NE), f32),
            jax.ShapeDtypeStruct((TOKENS, LATENT), bf16),
            jax.ShapeDtypeStruct((TOKENS, DIM), f32),
        ],
    )(X, W_down.astype(bf16), wr_hi, wr_lo, brow, W_core.astype(bf16),
      W_shared1.astype(bf16), W_shared2.astype(bf16),
      b_down.reshape(1, LATENT), b_core.reshape(1, DIM),
      b_shared1.reshape(1, LATENT), b_shared2.reshape(1, LATENT))

    tb3 = lambda n: pl.BlockSpec((TB3, n), lambda i: (i, 0))
    in_specs3 = [tb3(LATENT), tb3(2 * NE), tb3(LATENT), tb3(DIM),
                 full((8, 2 * NE)), full((DIM, LATENT)), full((1, DIM))]
    args3 = [g16, wg, sh16, core, counts, W_up.astype(bf16),
             b_up.reshape(1, DIM)]
    for e in range(NE):
        hd = expert_W1s[e].shape[0]
        in_specs3.append(full((hd, LATENT)))
        args3.append(expert_W1s[e].astype(bf16))
    for e in range(NE):
        hd = expert_b1s[e].shape[0]
        in_specs3.append(full((1, hd)))
        args3.append(expert_b1s[e].reshape(1, hd))
    for e in range(NE):
        hd = expert_W2s[e].shape[1]
        in_specs3.append(full((LATENT, hd)))
        args3.append(expert_W2s[e].astype(bf16))
    for e in range(NE):
        in_specs3.append(full((1, LATENT)))
        args3.append(expert_b2s[e].reshape(1, LATENT))

    out = pl.pallas_call(
        _stage3_body,
        grid=(TOKENS // TB3,),
        in_specs=in_specs3,
        out_specs=tb3(DIM),
        out_shape=jax.ShapeDtypeStruct((TOKENS, DIM), f32),
        scratch_shapes=[pltpu.VMEM((2 * NE, LATENT), f32)],
    )(*args3)

    return out.reshape(x.shape)


# speedup vs baseline: 2.9144x; 1.0102x over previous
import jax
import jax.numpy as jnp
from jax.experimental import pallas as pl
from jax.experimental.pallas import tpu as pltpu

DIM = 1024
LATENT = 512
NE = 8
TOKENS = 2048
TB = 1024
TB3 = 1024
COST_LAMBDA = 5e-07
_SQRT_HALF = 0.7071067811865476


def _gelu(x):
    x = x.astype(jnp.float32)
    return x * 0.5 * (1.0 + jax.lax.erf(x * _SQRT_HALF))


def _bdot(a16, b16):
    return jax.lax.dot_general(
        a16, b16, (((1,), (1,)), ((), ())), preferred_element_type=jnp.float32)


def _stage1_body(x_ref, wdown_ref, wrhi_ref, wrlo_ref, brow_ref, wcore_ref,
                 ws1_ref, ws2_ref, bdown_ref, bcore_ref, bs1_ref, bs2_ref,
                 g_ref, wg_ref, counts_ref, sh_ref, core_ref):
    xb = x_ref[...]
    xb16 = xb.astype(jnp.bfloat16)
    xlo16 = (xb - xb16.astype(jnp.float32)).astype(jnp.bfloat16)

    h = _bdot(xb16, wdown_ref[...]) + bdown_ref[...]
    g = _gelu(h)
    g16 = g.astype(jnp.bfloat16)
    g_ref[...] = g16

    logits = (_bdot(xb16, wrhi_ref[...])
              + (_bdot(xb16, wrlo_ref[...]) + _bdot(xlo16, wrhi_ref[...]))
              + brow_ref[...])
    mx = jnp.max(logits, axis=-1, keepdims=True)
    ex = jnp.exp(logits - mx)
    probs = ex / jnp.sum(ex, axis=-1, keepdims=True)

    iota = jax.lax.broadcasted_iota(jnp.int32, probs.shape, 1)
    m0 = jnp.max(probs, axis=-1, keepdims=True)
    e0 = jnp.min(jnp.where(probs >= m0, iota, NE), axis=-1, keepdims=True)
    oh0 = (iota == e0)
    probs1 = jnp.where(oh0, -1.0, probs)
    m1 = jnp.max(probs1, axis=-1, keepdims=True)
    e1 = jnp.min(jnp.where(probs1 >= m1, iota, NE), axis=-1, keepdims=True)
    oh1 = (iota == e1)

    ed = jnp.exp(m1 - m0)
    gate0 = 1.0 / (1.0 + ed)
    gate1 = ed * gate0

    w = gate0 * oh0.astype(jnp.float32) + gate1 * oh1.astype(jnp.float32)
    wg_ref[...] = jnp.concatenate(
        [w, jnp.broadcast_to(gate0, (TB, 4)), jnp.broadcast_to(gate1, (TB, 4))],
        axis=-1)

    a0 = jnp.max(oh0.astype(jnp.float32), axis=0, keepdims=True)
    a1 = jnp.max(oh1.astype(jnp.float32), axis=0, keepdims=True)
    cblk = jnp.broadcast_to(jnp.concatenate([a0, a1], axis=-1), (8, 2 * NE))

    @pl.when(pl.program_id(0) == 0)
    def _init():
        counts_ref[...] = cblk

    @pl.when(pl.program_id(0) > 0)
    def _acc():
        counts_ref[...] = jnp.maximum(counts_ref[...], cblk)

    core_ref[...] = _bdot(_gelu(xb).astype(jnp.bfloat16), wcore_ref[...]) \
        + bcore_ref[...]

    s1 = _gelu(_bdot(g16, ws1_ref[...]) + bs1_ref[...])
    sh_ref[...] = (_bdot(s1.astype(jnp.bfloat16), ws2_ref[...])
                   + bs2_ref[...]).astype(jnp.bfloat16)


def _stage3_body(*refs):
    (g_ref, wg_ref, sh_ref, core_ref, counts_ref, wup_ref, bup_ref) = refs[:7]
    w1_refs = refs[7:7 + NE]
    b1_refs = refs[7 + NE:7 + 2 * NE]
    w2_refs = refs[7 + 2 * NE:7 + 3 * NE]
    b2_refs = refs[7 + 3 * NE:7 + 4 * NE]
    out_ref = refs[-2]
    ca_ref = refs[-1]

    @pl.when(pl.program_id(0) == 0)
    def _constants():
        for e in range(NE):
            gb = _gelu(b1_refs[e][...]).astype(jnp.bfloat16)
            ca_ref[e:e + 1, :] = _bdot(gb, w2_refs[e][...]) + b2_refs[e][...]
        c = ca_ref[0:NE, :]
        act0 = counts_ref[0:1, 0:NE] > 0.5
        act1 = counts_ref[0:1, NE:2 * NE] > 0.5
        ca_ref[NE:NE + 1, :] = jnp.sum(jnp.where(act0.T, c, 0.0), axis=0,
                                       keepdims=True)
        ca_ref[NE + 1:NE + 2, :] = jnp.sum(jnp.where(act1.T, c, 0.0), axis=0,
                                           keepdims=True)

    g16 = g_ref[...]
    wg = wg_ref[...]
    moe = jnp.zeros((TB3, LATENT), jnp.float32)
    for e in range(NE):
        z1 = _gelu(_bdot(g16, w1_refs[e][...]) + b1_refs[e][...])
        z2 = _bdot(z1.astype(jnp.bfloat16), w2_refs[e][...]) + b2_refs[e][...]
        moe = moe + wg[:, e:e + 1] * (z2 - ca_ref[e:e + 1, :])
    gate0 = wg[:, NE:NE + 1]
    gate1 = wg[:, NE + 4:NE + 5]
    acc = (moe + gate0 * ca_ref[NE:NE + 1, :]
           + gate1 * ca_ref[NE + 1:NE + 2, :])
    t16 = (acc + 0.1 * sh_ref[...].astype(jnp.float32)).astype(jnp.bfloat16)
    out_ref[...] = _bdot(t16, wup_ref[...]) + bup_ref[...] + core_ref[...]


def kernel(x, W_down, b_down, W_up, b_up, expert_W1s, expert_b1s, expert_W2s,
           expert_b2s, W_shared1, b_shared1, W_shared2, b_shared2, W_router,
           b_router, W_core, b_core, expert_cost):
    f32 = jnp.float32
    bf16 = jnp.bfloat16
    X = x.reshape(TOKENS, DIM)
    nblk = TOKENS // TB

    brow = (b_router - COST_LAMBDA * expert_cost).reshape(1, NE)
    wr_hi = W_router.astype(bf16)
    wr_lo = (W_router - wr_hi.astype(f32)).astype(bf16)

    full = lambda shape: pl.BlockSpec(shape, lambda i: (0,) * len(shape))
    tb = lambda n: pl.BlockSpec((TB, n), lambda i: (i, 0))

    g16, wg, counts, sh16, core = pl.pallas_call(
        _stage1_body,
        grid=(nblk,),
        in_specs=[
            tb(DIM),
            full((LATENT, DIM)), full((NE, DIM)), full((NE, DIM)),
            full((1, NE)),
            full((DIM, DIM)), full((LATENT, LATENT)), full((LATENT, LATENT)),
            full((1, LATENT)), full((1, DIM)), full((1, LATENT)),
            full((1, LATENT)),
        ],
        out_specs=[
            tb(LATENT), tb(2 * NE),
            pl.BlockSpec((8, 2 * NE), lambda i: (0, 0)),
            tb(LATENT), tb(DIM),
        ],
        out_shape=[
            jax.ShapeDtypeStruct((TOKENS, LATENT), bf16),
            jax.ShapeDtypeStruct((TOKENS, 2 * NE), f32),
            jax.ShapeDtypeStruct((8, 2 * NE), f32),
            jax.ShapeDtypeStruct((TOKENS, LATENT), bf16),
            jax.ShapeDtypeStruct((TOKENS, DIM), f32),
        ],
    )(X, W_down.astype(bf16), wr_hi, wr_lo, brow, W_core.astype(bf16),
      W_shared1.astype(bf16), W_shared2.astype(bf16),
      b_down.reshape(1, LATENT), b_core.reshape(1, DIM),
      b_shared1.reshape(1, LATENT), b_shared2.reshape(1, LATENT))

    tb3 = lambda n: pl.BlockSpec((TB3, n), lambda i: (i, 0))
    in_specs3 = [tb3(LATENT), tb3(2 * NE), tb3(LATENT), tb3(DIM),
                 full((8, 2 * NE)), full((DIM, LATENT)), full((1, DIM))]
    args3 = [g16, wg, sh16, core, counts, W_up.astype(bf16),
             b_up.reshape(1, DIM)]
    for e in range(NE):
        hd = expert_W1s[e].shape[0]
        in_specs3.append(full((hd, LATENT)))
        args3.append(expert_W1s[e].astype(bf16))
    for e in range(NE):
        hd = expert_b1s[e].shape[0]
        in_specs3.append(full((1, hd)))
        args3.append(expert_b1s[e].reshape(1, hd))
    for e in range(NE):
        hd = expert_W2s[e].shape[1]
        in_specs3.append(full((LATENT, hd)))
        args3.append(expert_W2s[e].astype(bf16))
    for e in range(NE):
        in_specs3.append(full((1, LATENT)))
        args3.append(expert_b2s[e].reshape(1, LATENT))

    out = pl.pallas_call(
        _stage3_body,
        grid=(TOKENS // TB3,),
        in_specs=in_specs3,
        out_specs=tb3(DIM),
        out_shape=jax.ShapeDtypeStruct((TOKENS, DIM), f32),
        scratch_shapes=[pltpu.VMEM((2 * NE, LATENT), f32)],
    )(*args3)

    return out.reshape(x.shape)


# speedup vs baseline: 2.9445x; 1.0103x over previous
import jax
import jax.numpy as jnp
from jax.experimental import pallas as pl
from jax.experimental.pallas import tpu as pltpu

DIM = 1024
LATENT = 512
NE = 8
TOKENS = 2048
TB = 1024
TB3 = 1024
COST_LAMBDA = 5e-07
_SQRT_HALF = 0.7071067811865476


def _gelu(x):
    x = x.astype(jnp.float32)
    return x * 0.5 * (1.0 + jax.lax.erf(x * _SQRT_HALF))


def _bdot(a16, b16):
    return jax.lax.dot_general(
        a16, b16, (((1,), (1,)), ((), ())), preferred_element_type=jnp.float32)


def _stage1_body(x_ref, wdown_ref, wrhi_ref, wrlo_ref, brow_ref, wcore_ref,
                 ws1_ref, ws2_ref, bdown_ref, bcore_ref, bs1_ref, bs2_ref,
                 g_ref, wg_ref, counts_ref, sh_ref, core_ref):
    xb = x_ref[...]
    xb16 = xb.astype(jnp.bfloat16)
    xlo16 = (xb - xb16.astype(jnp.float32)).astype(jnp.bfloat16)

    h = _bdot(xb16, wdown_ref[...]) + bdown_ref[...]
    g = _gelu(h)
    g16 = g.astype(jnp.bfloat16)
    g_ref[...] = g16

    logits = (_bdot(xb16, wrhi_ref[...])
              + (_bdot(xb16, wrlo_ref[...]) + _bdot(xlo16, wrhi_ref[...]))
              + brow_ref[...])
    mx = jnp.max(logits, axis=-1, keepdims=True)
    ex = jnp.exp(logits - mx)
    probs = ex / jnp.sum(ex, axis=-1, keepdims=True)

    iota = jax.lax.broadcasted_iota(jnp.int32, probs.shape, 1)
    m0 = jnp.max(probs, axis=-1, keepdims=True)
    e0 = jnp.min(jnp.where(probs >= m0, iota, NE), axis=-1, keepdims=True)
    oh0 = (iota == e0)
    probs1 = jnp.where(oh0, -1.0, probs)
    m1 = jnp.max(probs1, axis=-1, keepdims=True)
    e1 = jnp.min(jnp.where(probs1 >= m1, iota, NE), axis=-1, keepdims=True)
    oh1 = (iota == e1)

    ed = jnp.exp(m1 - m0)
    gate0 = 1.0 / (1.0 + ed)
    gate1 = ed * gate0

    w = gate0 * oh0.astype(jnp.float32) + gate1 * oh1.astype(jnp.float32)
    wg_ref[...] = jnp.concatenate(
        [w, jnp.broadcast_to(gate0, (TB, 4)), jnp.broadcast_to(gate1, (TB, 4))],
        axis=-1)

    a0 = jnp.max(oh0.astype(jnp.float32), axis=0, keepdims=True)
    a1 = jnp.max(oh1.astype(jnp.float32), axis=0, keepdims=True)
    cblk = jnp.broadcast_to(jnp.concatenate([a0, a1], axis=-1), (8, 2 * NE))

    @pl.when(pl.program_id(0) == 0)
    def _init():
        counts_ref[...] = cblk

    @pl.when(pl.program_id(0) > 0)
    def _acc():
        counts_ref[...] = jnp.maximum(counts_ref[...], cblk)

    core_ref[...] = (_bdot(_gelu(xb).astype(jnp.bfloat16), wcore_ref[...])
                     + bcore_ref[...]).astype(jnp.bfloat16)

    s1 = _gelu(_bdot(g16, ws1_ref[...]) + bs1_ref[...])
    sh_ref[...] = (_bdot(s1.astype(jnp.bfloat16), ws2_ref[...])
                   + bs2_ref[...]).astype(jnp.bfloat16)


def _stage3_body(*refs):
    (g_ref, wg_ref, sh_ref, core_ref, counts_ref, wup_ref, bup_ref) = refs[:7]
    w1_refs = refs[7:7 + NE]
    b1_refs = refs[7 + NE:7 + 2 * NE]
    w2_refs = refs[7 + 2 * NE:7 + 3 * NE]
    b2_refs = refs[7 + 3 * NE:7 + 4 * NE]
    out_ref = refs[-2]
    ca_ref = refs[-1]

    @pl.when(pl.program_id(0) == 0)
    def _constants():
        for e in range(NE):
            gb = _gelu(b1_refs[e][...]).astype(jnp.bfloat16)
            ca_ref[e:e + 1, :] = _bdot(gb, w2_refs[e][...]) + b2_refs[e][...]
        c = ca_ref[0:NE, :]
        act0 = counts_ref[0:1, 0:NE] > 0.5
        act1 = counts_ref[0:1, NE:2 * NE] > 0.5
        ca_ref[NE:NE + 1, :] = jnp.sum(jnp.where(act0.T, c, 0.0), axis=0,
                                       keepdims=True)
        ca_ref[NE + 1:NE + 2, :] = jnp.sum(jnp.where(act1.T, c, 0.0), axis=0,
                                           keepdims=True)

    g16 = g_ref[...]
    wg = wg_ref[...]
    moe = jnp.zeros((TB3, LATENT), jnp.float32)
    for e in range(NE):
        z1 = _gelu(_bdot(g16, w1_refs[e][...]) + b1_refs[e][...])
        z2 = _bdot(z1.astype(jnp.bfloat16), w2_refs[e][...]) + b2_refs[e][...]
        moe = moe + wg[:, e:e + 1] * (z2 - ca_ref[e:e + 1, :])
    gate0 = wg[:, NE:NE + 1]
    gate1 = wg[:, NE + 4:NE + 5]
    acc = (moe + gate0 * ca_ref[NE:NE + 1, :]
           + gate1 * ca_ref[NE + 1:NE + 2, :])
    t16 = (acc + 0.1 * sh_ref[...].astype(jnp.float32)).astype(jnp.bfloat16)
    out_ref[...] = (_bdot(t16, wup_ref[...]) + bup_ref[...]
                    + core_ref[...].astype(jnp.float32))


def kernel(x, W_down, b_down, W_up, b_up, expert_W1s, expert_b1s, expert_W2s,
           expert_b2s, W_shared1, b_shared1, W_shared2, b_shared2, W_router,
           b_router, W_core, b_core, expert_cost):
    f32 = jnp.float32
    bf16 = jnp.bfloat16
    X = x.reshape(TOKENS, DIM)
    nblk = TOKENS // TB

    brow = (b_router - COST_LAMBDA * expert_cost).reshape(1, NE)
    wr_hi = W_router.astype(bf16)
    wr_lo = (W_router - wr_hi.astype(f32)).astype(bf16)

    full = lambda shape: pl.BlockSpec(shape, lambda i: (0,) * len(shape))
    tb = lambda n: pl.BlockSpec((TB, n), lambda i: (i, 0))

    g16, wg, counts, sh16, core = pl.pallas_call(
        _stage1_body,
        grid=(nblk,),
        in_specs=[
            tb(DIM),
            full((LATENT, DIM)), full((NE, DIM)), full((NE, DIM)),
            full((1, NE)),
            full((DIM, DIM)), full((LATENT, LATENT)), full((LATENT, LATENT)),
            full((1, LATENT)), full((1, DIM)), full((1, LATENT)),
            full((1, LATENT)),
        ],
        out_specs=[
            tb(LATENT), tb(2 * NE),
            pl.BlockSpec((8, 2 * NE), lambda i: (0, 0)),
            tb(LATENT), tb(DIM),
        ],
        out_shape=[
            jax.ShapeDtypeStruct((TOKENS, LATENT), bf16),
            jax.ShapeDtypeStruct((TOKENS, 2 * NE), f32),
            jax.ShapeDtypeStruct((8, 2 * NE), f32),
            jax.ShapeDtypeStruct((TOKENS, LATENT), bf16),
            jax.ShapeDtypeStruct((TOKENS, DIM), bf16),
        ],
    )(X, W_down.astype(bf16), wr_hi, wr_lo, brow, W_core.astype(bf16),
      W_shared1.astype(bf16), W_shared2.astype(bf16),
      b_down.reshape(1, LATENT), b_core.reshape(1, DIM),
      b_shared1.reshape(1, LATENT), b_shared2.reshape(1, LATENT))

    tb3 = lambda n: pl.BlockSpec((TB3, n), lambda i: (i, 0))
    in_specs3 = [tb3(LATENT), tb3(2 * NE), tb3(LATENT), tb3(DIM),
                 full((8, 2 * NE)), full((DIM, LATENT)), full((1, DIM))]
    args3 = [g16, wg, sh16, core, counts, W_up.astype(bf16),
             b_up.reshape(1, DIM)]
    for e in range(NE):
        hd = expert_W1s[e].shape[0]
        in_specs3.append(full((hd, LATENT)))
        args3.append(expert_W1s[e].astype(bf16))
    for e in range(NE):
        hd = expert_b1s[e].shape[0]
        in_specs3.append(full((1, hd)))
        args3.append(expert_b1s[e].reshape(1, hd))
    for e in range(NE):
        hd = expert_W2s[e].shape[1]
        in_specs3.append(full((LATENT, hd)))
        args3.append(expert_W2s[e].astype(bf16))
    for e in range(NE):
        in_specs3.append(full((1, LATENT)))
        args3.append(expert_b2s[e].reshape(1, LATENT))

    out = pl.pallas_call(
        _stage3_body,
        grid=(TOKENS // TB3,),
        in_specs=in_specs3,
        out_specs=tb3(DIM),
        out_shape=jax.ShapeDtypeStruct((TOKENS, DIM), f32),
        scratch_shapes=[pltpu.VMEM((2 * NE, LATENT), f32)],
    )(*args3)

    return out.reshape(x.shape)
